# Initial kernel scaffold; baseline (speedup 1.0000x reference)
import jax
import jax.numpy as jnp
from jax.experimental import pallas as pl


def kernel(x, conv1_w, conv1_b, conv2_w, conv2_b, dense_w, dense_b):
    raise NotImplementedError("write your pallas kernel here")


# speedup vs baseline: 4.5893x; 4.5893x over previous
import numpy as np
import jax
import jax.numpy as jnp
from jax.experimental import pallas as pl
from jax.experimental.pallas import tpu as pltpu

_H = 224
_H1 = _H - 2
_POOL1 = 512
_H2 = (_POOL1 - 3) // 2 + 1
_P = 16
_D = _P * _P
_CO = 8
_VMEM_LIMIT = 48 * 1024 * 1024


def _pool_matrix(in_size, out_size):
    P = np.zeros((out_size, in_size), np.float32)
    for i in range(out_size):
        s = (i * in_size) // out_size
        e = -(-((i + 1) * in_size) // out_size)
        P[i, s:e] = 1.0 / (e - s)
    return P


def _build_operators():
    P1 = _pool_matrix(_H1, _POOL1)
    P2 = _pool_matrix(_H2, _P)
    Ls = []
    for d in range(3):
        R = np.zeros((_H2, _POOL1), np.float32)
        R[np.arange(_H2), 2 * np.arange(_H2) + d] = 1.0
        Ls.append(P2 @ R @ P1)
    L_all = np.concatenate(Ls, axis=0)
    rs = L_all.sum(axis=1)
    blk = np.outer(rs, rs)
    return L_all, blk


_L_ALL, _BIAS_BLK = _build_operators()


def _fused_body(w1_ref, k2_ref, x_ref, l_ref, lt_ref, zb_ref, o_ref):
    L = l_ref[...]
    Lt = lt_ref[...]

    accs = [None, None, None]
    for cp in range(3):
        for a in range(3):
            for b in range(3):
                sl = x_ref[0, cp, a:a + _H1, b:b + _H1]
                for c in range(3):
                    w = w1_ref[((c * 3 + cp) * 3 + a) * 3 + b]
                    accs[c] = w * sl if accs[c] is None else accs[c] + w * sl

    ublk = []
    for c in range(3):
        T = jnp.dot(L, accs[c], preferred_element_type=jnp.float32)
        U = jnp.dot(T, Lt, preferred_element_type=jnp.float32)
        ublk.append([[U[dh * _P:(dh + 1) * _P, dw * _P:(dw + 1) * _P]
                      for dw in range(3)] for dh in range(3)])

    for o in range(_CO):
        z = zb_ref[o]
        for c in range(3):
            for dh in range(3):
                for dw in range(3):
                    k = k2_ref[((o * 3 + c) * 3 + dh) * 3 + dw]
                    z = z + k * ublk[c][dh][dw]
        o_ref[0, o] = z


def _dense_body(a_ref, w_ref, b_ref, o_ref):
    o_ref[...] = (jnp.dot(a_ref[...], w_ref[...],
                          preferred_element_type=jnp.float32) + b_ref[...])


def kernel(x, conv1_w, conv1_b, conv2_w, conv2_b, dense_w, dense_b):
    N = x.shape[0]
    L = jnp.asarray(_L_ALL)
    Lt = jnp.asarray(_L_ALL.T.copy())

    k2 = conv2_w.astype(jnp.float32)
    b1 = conv1_b.astype(jnp.float32)
    blk4 = jnp.asarray(_BIAS_BLK.reshape(3, _P, 3, _P))
    zbias = (conv2_b.astype(jnp.float32)[:, None, None]
             + jnp.einsum('ochw,c,hiwj->oij', k2, b1, blk4))

    w1_flat = conv1_w.astype(jnp.float32).reshape(-1)
    k2_flat = k2.reshape(-1)

    z = pl.pallas_call(
        _fused_body,
        grid=(N,),
        in_specs=[
            pl.BlockSpec(memory_space=pltpu.MemorySpace.SMEM),
            pl.BlockSpec(memory_space=pltpu.MemorySpace.SMEM),
            pl.BlockSpec((1, 3, _H, _H), lambda n: (n, 0, 0, 0)),
            pl.BlockSpec((48, _H1), lambda n: (0, 0)),
            pl.BlockSpec((_H1, 48), lambda n: (0, 0)),
            pl.BlockSpec((_CO, _P, _P), lambda n: (0, 0, 0)),
        ],
        out_specs=pl.BlockSpec((1, _CO, _P, _P), lambda n: (n, 0, 0, 0)),
        out_shape=jax.ShapeDtypeStruct((N, _CO, _P, _P), jnp.float32),
        compiler_params=pltpu.CompilerParams(
            dimension_semantics=("parallel",),
            vmem_limit_bytes=_VMEM_LIMIT),
    )(w1_flat, k2_flat, x.astype(jnp.float32), L, Lt, zbias)

    flat = z.reshape(N * _CO, _D)
    wt = dense_w.astype(jnp.float32).T
    bias2d = dense_b.astype(jnp.float32).reshape(1, _D)
    M = N * _CO
    tm = M // 2
    out = pl.pallas_call(
        _dense_body,
        grid=(2,),
        in_specs=[
            pl.BlockSpec((tm, _D), lambda i: (i, 0)),
            pl.BlockSpec((_D, _D), lambda i: (0, 0)),
            pl.BlockSpec((1, _D), lambda i: (0, 0)),
        ],
        out_specs=pl.BlockSpec((tm, _D), lambda i: (i, 0)),
        out_shape=jax.ShapeDtypeStruct((M, _D), jnp.float32),
        compiler_params=pltpu.CompilerParams(
            dimension_semantics=("parallel",),
            vmem_limit_bytes=_VMEM_LIMIT),
    )(flat, wt, bias2d)
    return out.reshape(N, _CO, _D)


# speedup vs baseline: 17.1156x; 3.7295x over previous
import numpy as np
import jax
import jax.numpy as jnp
from jax.experimental import pallas as pl
from jax.experimental.pallas import tpu as pltpu

_H = 224
_H1 = _H - 2
_POOL1 = 512
_H2 = (_POOL1 - 3) // 2 + 1
_P = 16
_D = _P * _P
_CO = 8
_VMEM_LIMIT = 48 * 1024 * 1024


def _pool_matrix(in_size, out_size):
    P = np.zeros((out_size, in_size), np.float32)
    for i in range(out_size):
        s = (i * in_size) // out_size
        e = -(-((i + 1) * in_size) // out_size)
        P[i, s:e] = 1.0 / (e - s)
    return P


def _build_operators():
    P1 = _pool_matrix(_H1, _POOL1)
    P2 = _pool_matrix(_H2, _P)
    Ls = []
    for d in range(3):
        R = np.zeros((_H2, _POOL1), np.float32)
        R[np.arange(_H2), 2 * np.arange(_H2) + d] = 1.0
        Ls.append(P2 @ R @ P1)
    L_all = np.concatenate(Ls, axis=0)
    emb = np.zeros((3, 48, _H), np.float32)
    for a in range(3):
        emb[a, :, a:a + _H1] = L_all
    Lrow = emb.reshape(144, _H)
    Lcolt = np.ascontiguousarray(np.transpose(emb, (0, 2, 1)))
    rs = L_all.sum(axis=1)
    blk = np.outer(rs, rs)
    return Lrow, Lcolt, blk


_LROW, _LCOLT, _BIAS_BLK = _build_operators()


def _fused_body(w1_ref, k2_ref, x_ref, lrow_ref, lcolt_ref, zb_ref, o_ref):
    Lrow = lrow_ref[...]

    A = [jnp.dot(Lrow, x_ref[0, cp], preferred_element_type=jnp.float32)
         for cp in range(3)]

    ublk = []
    for c in range(3):
        U = None
        for b in range(3):
            Bacc = None
            for cp in range(3):
                for a in range(3):
                    w = w1_ref[((c * 3 + cp) * 3 + a) * 3 + b]
                    term = w * A[cp][48 * a:48 * a + 48, :]
                    Bacc = term if Bacc is None else Bacc + term
            Ub = jnp.dot(Bacc, lcolt_ref[b],
                         preferred_element_type=jnp.float32)
            U = Ub if U is None else U + Ub
        ublk.append([[U[dh * _P:(dh + 1) * _P, dw * _P:(dw + 1) * _P]
                      for dw in range(3)] for dh in range(3)])

    for o in range(_CO):
        z = zb_ref[o]
        for c in range(3):
            for dh in range(3):
                for dw in range(3):
                    k = k2_ref[((o * 3 + c) * 3 + dh) * 3 + dw]
                    z = z + k * ublk[c][dh][dw]
        o_ref[0, o] = z


def _dense_body(a_ref, w_ref, b_ref, o_ref):
    o_ref[...] = (jnp.dot(a_ref[...], w_ref[...],
                          preferred_element_type=jnp.float32) + b_ref[...])


def kernel(x, conv1_w, conv1_b, conv2_w, conv2_b, dense_w, dense_b):
    N = x.shape[0]
    Lrow = jnp.asarray(_LROW)
    Lcolt = jnp.asarray(_LCOLT)

    k2 = conv2_w.astype(jnp.float32)
    b1 = conv1_b.astype(jnp.float32)
    blk4 = jnp.asarray(_BIAS_BLK.reshape(3, _P, 3, _P))
    zbias = (conv2_b.astype(jnp.float32)[:, None, None]
             + jnp.einsum('ochw,c,hiwj->oij', k2, b1, blk4))

    w1_flat = conv1_w.astype(jnp.float32).reshape(-1)
    k2_flat = k2.reshape(-1)

    z = pl.pallas_call(
        _fused_body,
        grid=(N,),
        in_specs=[
            pl.BlockSpec(memory_space=pltpu.MemorySpace.SMEM),
            pl.BlockSpec(memory_space=pltpu.MemorySpace.SMEM),
            pl.BlockSpec((1, 3, _H, _H), lambda n: (n, 0, 0, 0)),
            pl.BlockSpec((144, _H), lambda n: (0, 0)),
            pl.BlockSpec((3, _H, 48), lambda n: (0, 0, 0)),
            pl.BlockSpec((_CO, _P, _P), lambda n: (0, 0, 0)),
        ],
        out_specs=pl.BlockSpec((1, _CO, _P, _P), lambda n: (n, 0, 0, 0)),
        out_shape=jax.ShapeDtypeStruct((N, _CO, _P, _P), jnp.float32),
        compiler_params=pltpu.CompilerParams(
            dimension_semantics=("parallel",),
            vmem_limit_bytes=_VMEM_LIMIT),
    )(w1_flat, k2_flat, x.astype(jnp.float32), Lrow, Lcolt, zbias)

    flat = z.reshape(N * _CO, _D)
    wt = dense_w.astype(jnp.float32).T
    bias2d = dense_b.astype(jnp.float32).reshape(1, _D)
    M = N * _CO
    tm = M // 2
    out = pl.pallas_call(
        _dense_body,
        grid=(2,),
        in_specs=[
            pl.BlockSpec((tm, _D), lambda i: (i, 0)),
            pl.BlockSpec((_D, _D), lambda i: (0, 0)),
            pl.BlockSpec((1, _D), lambda i: (0, 0)),
        ],
        out_specs=pl.BlockSpec((tm, _D), lambda i: (i, 0)),
        out_shape=jax.ShapeDtypeStruct((M, _D), jnp.float32),
        compiler_params=pltpu.CompilerParams(
            dimension_semantics=("parallel",),
            vmem_limit_bytes=_VMEM_LIMIT),
    )(flat, wt, bias2d)
    return out.reshape(N, _CO, _D)
